# baseline (speedup 1.0000x reference)
import functools

import jax
import jax.numpy as jnp
from jax import lax
from jax.experimental import pallas as pl
from jax.experimental.pallas import tpu as pltpu
from jax.experimental.pallas import tpu_sc as plsc

N = 1_000_000
NPAIR = 262_144

NT = 16
SEG = 62_528
NSTAGE = 8
STG = SEG // NSTAGE

PAIRS_PER_TILE = NPAIR // NT


def _sc_scatter_add(inp, idx, val):
    mesh = plsc.VectorSubcoreMesh(core_axis_name="c", subcore_axis_name="s",
                                  num_cores=1)

    @functools.partial(
        pl.kernel,
        mesh=mesh,
        out_type=jax.ShapeDtypeStruct((N,), jnp.int32),
        scratch_types=[
            pltpu.VMEM_SHARED((N,), jnp.int32),
            pltpu.VMEM((PAIRS_PER_TILE,), jnp.int32),
            pltpu.VMEM((PAIRS_PER_TILE,), jnp.int32),
            pltpu.VMEM((STG,), jnp.int32),
            pltpu.VMEM((STG,), jnp.int32),
            pltpu.SemaphoreType.DMA,
            pltpu.SemaphoreType.DMA,
            pltpu.SemaphoreType.DMA,
        ],
    )
    def k(in_hbm, idx_hbm, val_hbm, out_hbm, acc_sh, idx_v, val_v,
          stage_a, stage_b, sem_pair, sem_pull, sem_push):
        tid = lax.axis_index("s")
        bufs = (stage_a, stage_b)

        def off(c):
            o = tid * SEG + c * STG
            if c == NSTAGE - 1:
                o = jnp.minimum(o, N - STG)
            return o

        def pipeline(src, dst, first_extra=None):
            pulls = [None] * NSTAGE
            pushes = [None] * NSTAGE
            pulls[0] = pltpu.async_copy(src.at[pl.ds(off(0), STG)], bufs[0],
                                        sem_pull)
            if first_extra is not None:
                first_extra()
            pulls[1] = pltpu.async_copy(src.at[pl.ds(off(1), STG)], bufs[1],
                                        sem_pull)
            for c in range(NSTAGE):
                pulls[c].wait()
                pushes[c] = pltpu.async_copy(bufs[c % 2],
                                             dst.at[pl.ds(off(c), STG)],
                                             sem_push)
                if c + 2 < NSTAGE:
                    pushes[c].wait()
                    pulls[c + 2] = pltpu.async_copy(
                        src.at[pl.ds(off(c + 2), STG)], bufs[c % 2], sem_pull)
            pushes[NSTAGE - 2].wait()
            pushes[NSTAGE - 1].wait()

        cps = []

        def load_pairs():
            pair0 = tid * PAIRS_PER_TILE
            cps.append(pltpu.async_copy(
                idx_hbm.at[pl.ds(pair0, PAIRS_PER_TILE)], idx_v, sem_pair))
            cps.append(pltpu.async_copy(
                val_hbm.at[pl.ds(pair0, PAIRS_PER_TILE)], val_v, sem_pair))

        pipeline(in_hbm, acc_sh, first_extra=load_pairs)

        for cp in cps:
            cp.wait()
        plsc.subcore_barrier()

        pltpu.sync_copy(val_v, acc_sh.at[idx_v], add=True)

        plsc.subcore_barrier()

        pipeline(acc_sh, out_hbm)

    return k(inp, idx, val)


def kernel(input, index, value):
    return _sc_scatter_add(input, index, value)


# speedup vs baseline: 1.0401x; 1.0401x over previous
import functools

import jax
import jax.numpy as jnp
from jax import lax
from jax.experimental import pallas as pl
from jax.experimental.pallas import tpu as pltpu
from jax.experimental.pallas import tpu_sc as plsc

N = 1_000_000
NPAIR = 262_144

NT = 16
SEG = 62_528
NSTAGE = 4
STG = SEG // NSTAGE

PAIRS_PER_TILE = NPAIR // NT


def _sc_scatter_add(inp, idx, val):
    mesh = plsc.VectorSubcoreMesh(core_axis_name="c", subcore_axis_name="s",
                                  num_cores=1)

    @functools.partial(
        pl.kernel,
        mesh=mesh,
        out_type=jax.ShapeDtypeStruct((N,), jnp.int32),
        scratch_types=[
            pltpu.VMEM_SHARED((N,), jnp.int32),
            pltpu.VMEM((PAIRS_PER_TILE,), jnp.int32),
            pltpu.VMEM((PAIRS_PER_TILE,), jnp.int32),
            pltpu.VMEM((STG,), jnp.int32),
            pltpu.VMEM((STG,), jnp.int32),
            pltpu.SemaphoreType.DMA,
            pltpu.SemaphoreType.DMA,
            pltpu.SemaphoreType.DMA,
        ],
    )
    def k(in_hbm, idx_hbm, val_hbm, out_hbm, acc_sh, idx_v, val_v,
          stage_a, stage_b, sem_pair, sem_pull, sem_push):
        tid = lax.axis_index("s")
        bufs = (stage_a, stage_b)

        def off(c):
            o = tid * SEG + c * STG
            if c == NSTAGE - 1:
                o = jnp.minimum(o, N - STG)
            return o

        def pipeline(src, dst, first_extra=None):
            pulls = [None] * NSTAGE
            pushes = [None] * NSTAGE
            pulls[0] = pltpu.async_copy(src.at[pl.ds(off(0), STG)], bufs[0],
                                        sem_pull)
            if first_extra is not None:
                first_extra()
            pulls[1] = pltpu.async_copy(src.at[pl.ds(off(1), STG)], bufs[1],
                                        sem_pull)
            for c in range(NSTAGE):
                pulls[c].wait()
                pushes[c] = pltpu.async_copy(bufs[c % 2],
                                             dst.at[pl.ds(off(c), STG)],
                                             sem_push)
                if c + 2 < NSTAGE:
                    pushes[c].wait()
                    pulls[c + 2] = pltpu.async_copy(
                        src.at[pl.ds(off(c + 2), STG)], bufs[c % 2], sem_pull)
            pushes[NSTAGE - 2].wait()
            pushes[NSTAGE - 1].wait()

        cps = []

        def load_pairs():
            pair0 = tid * PAIRS_PER_TILE
            cps.append(pltpu.async_copy(
                idx_hbm.at[pl.ds(pair0, PAIRS_PER_TILE)], idx_v, sem_pair))
            cps.append(pltpu.async_copy(
                val_hbm.at[pl.ds(pair0, PAIRS_PER_TILE)], val_v, sem_pair))

        pipeline(in_hbm, acc_sh, first_extra=load_pairs)

        for cp in cps:
            cp.wait()
        plsc.subcore_barrier()

        pltpu.sync_copy(val_v, acc_sh.at[idx_v], add=True)

        plsc.subcore_barrier()

        wbufs = (stage_a, stage_b, idx_v.at[pl.ds(0, STG)],
                 val_v.at[pl.ds(0, STG)])
        pulls = [pltpu.async_copy(acc_sh.at[pl.ds(off(c), STG)], wbufs[c],
                                  sem_pull)
                 for c in range(NSTAGE)]
        pushes = []
        for c in range(NSTAGE):
            pulls[c].wait()
            pushes.append(pltpu.async_copy(wbufs[c],
                                           out_hbm.at[pl.ds(off(c), STG)],
                                           sem_push))
        for p in pushes:
            p.wait()

    return k(inp, idx, val)


def kernel(input, index, value):
    return _sc_scatter_add(input, index, value)


# speedup vs baseline: 1.0401x; 1.0000x over previous
import functools

import jax
import jax.numpy as jnp
from jax import lax
from jax.experimental import pallas as pl
from jax.experimental.pallas import tpu as pltpu
from jax.experimental.pallas import tpu_sc as plsc

N = 1_000_000
NPAIR = 262_144

NT = 16
SEG = 62_528
NSTAGE = 4
STG = SEG // NSTAGE

PAIRS_PER_TILE = NPAIR // NT


def _sc_scatter_add(inp, idx, val):
    mesh = plsc.VectorSubcoreMesh(core_axis_name="c", subcore_axis_name="s",
                                  num_cores=1)

    @functools.partial(
        pl.kernel,
        mesh=mesh,
        out_type=jax.ShapeDtypeStruct((N,), jnp.int32),
        scratch_types=[
            pltpu.VMEM_SHARED((N,), jnp.int32),
            pltpu.VMEM((PAIRS_PER_TILE,), jnp.int32),
            pltpu.VMEM((PAIRS_PER_TILE,), jnp.int32),
            pltpu.VMEM((STG,), jnp.int32),
            pltpu.VMEM((STG,), jnp.int32),
            pltpu.SemaphoreType.DMA,
            pltpu.SemaphoreType.DMA,
            pltpu.SemaphoreType.DMA,
        ],
    )
    def k(in_hbm, idx_hbm, val_hbm, out_hbm, acc_sh, idx_v, val_v,
          stage_a, stage_b, sem_pair, sem_pull, sem_push):
        tid = lax.axis_index("s")
        bufs = (stage_a, stage_b)

        def off(c):
            o = tid * SEG + c * STG
            if c == NSTAGE - 1:
                o = jnp.minimum(o, N - STG)
            return o

        def ipull(c, buf):
            return pltpu.async_copy(in_hbm.at[pl.ds(off(c), STG)], buf,
                                    sem_pull)

        def ipush(c, buf):
            return pltpu.async_copy(buf, acc_sh.at[pl.ds(off(c), STG)],
                                    sem_push)

        pair0 = tid * PAIRS_PER_TILE
        p0 = ipull(0, stage_a)
        p1 = ipull(1, stage_b)
        p0.wait()
        s0 = ipush(0, stage_a)
        s0.wait()
        p2 = ipull(2, stage_a)
        cp_idx = pltpu.async_copy(
            idx_hbm.at[pl.ds(pair0, PAIRS_PER_TILE)], idx_v, sem_pair)
        p1.wait()
        s1 = ipush(1, stage_b)
        s1.wait()
        p3 = ipull(3, stage_b)
        cp_val = pltpu.async_copy(
            val_hbm.at[pl.ds(pair0, PAIRS_PER_TILE)], val_v, sem_pair)
        p2.wait()
        s2 = ipush(2, stage_a)
        p3.wait()
        s3 = ipush(3, stage_b)
        s2.wait()
        s3.wait()
        cp_idx.wait()
        cp_val.wait()
        plsc.subcore_barrier()

        pltpu.sync_copy(val_v, acc_sh.at[idx_v], add=True)

        plsc.subcore_barrier()

        wbufs = (stage_a, stage_b, idx_v.at[pl.ds(0, STG)],
                 val_v.at[pl.ds(0, STG)])
        pulls = [pltpu.async_copy(acc_sh.at[pl.ds(off(c), STG)], wbufs[c],
                                  sem_pull)
                 for c in range(NSTAGE)]
        pushes = []
        for c in range(NSTAGE):
            pulls[c].wait()
            pushes.append(pltpu.async_copy(wbufs[c],
                                           out_hbm.at[pl.ds(off(c), STG)],
                                           sem_push))
        for p in pushes:
            p.wait()

    return k(inp, idx, val)


def kernel(input, index, value):
    return _sc_scatter_add(input, index, value)
